# baseline (speedup 1.0000x reference)
import functools

import jax
import jax.numpy as jnp
from jax import lax
from jax.experimental import pallas as pl
from jax.experimental.pallas import tpu as pltpu
from jax.experimental.pallas import tpu_sc as plsc

_P = 16
_N, _C, _S, _K = 16, 256, 8, 1024
_NW = 32
_ROWS = _N * _S
_RPW = _ROWS // _NW
_CH = 32
_NCHUNK = _C // _CH
_U = 8


def _splat_lane(vec, u):
    idx = jnp.full((16, 1), u, jnp.int32)
    dnums = lax.GatherDimensionNumbers(
        offset_dims=(), collapsed_slice_dims=(0,), start_index_map=(0,))
    return lax.gather(vec, idx, dnums, (1,),
                      mode=lax.GatherScatterMode.PROMISE_IN_BOUNDS)


def _sc_body(feats_hbm, lab_hbm, vm_hbm, out_hbm,
             fbufA, fbufB, lab_v, vm_v,
             s0b0, s0b1, m0b0, m0b1, m0b2, m0b3,
             s1b0, s1b1, m1b0, m1b1, m1b2, m1b3,
             cntb0, cntb1, patchb0, patchb1,
             recip_v, mask_v, outbA, outbB,
             semA, semB, semOA, semOB):
    cid = lax.axis_index("c")
    sid = lax.axis_index("s")
    wid = sid * 2 + cid
    iota = lax.iota(jnp.int32, 16)
    iota16 = iota + 16
    zero16 = jnp.zeros((16,), jnp.float32)
    neg100 = jnp.full((16,), -100.0, jnp.float32)
    ones16 = jnp.ones((16,), jnp.float32)

    sum0 = (s0b0, s0b1)
    sum1 = (s1b0, s1b1)
    max0 = (m0b0, m0b1, m0b2, m0b3)
    max1 = (m1b0, m1b1, m1b2, m1b3)

    for ri in range(_RPW):
        row = wid * _RPW + ri
        n0 = row // _S
        s0 = lax.rem(row, _S)

        pltpu.async_copy(
            feats_hbm.at[n0, pl.ds(0, _CH), s0, :], fbufA, semA)

        pltpu.sync_copy(lab_hbm.at[n0, s0], lab_v)
        pltpu.sync_copy(vm_hbm.at[n0, s0], vm_v)

        for r in range(16):
            cntb0[r, :] = zero16
            cntb1[r, :] = zero16
            patchb0[r, :] = zero16
            patchb1[r, :] = zero16

        def cbody(j, carry):
            for u in range(4):
                jj = j * 4 + u
                labv = lab_v[pl.ds(jj * 16, 16)]
                vmv = vm_v[pl.ds(jj * 16, 16)]
                cb = (cntb0, cntb1)[u % 2]
                pb = (patchb0, patchb1)[u % 2]
                plsc.addupdate_scatter(cb, [iota, labv], vmv)
                plsc.addupdate_scatter(pb, [iota, labv], ones16)
            return carry
        lax.fori_loop(0, _K // 64, cbody, 0)

        cnt = cntb0[0, :] + cntb1[0, :]
        patch = patchb0[0, :] + patchb1[0, :]
        for r in range(1, 16):
            cnt = cnt + cntb0[r, :] + cntb1[r, :]
            patch = patch + patchb0[r, :] + patchb1[r, :]
        recip_v[:] = 1.0 / jnp.maximum(cnt, 1.0)
        mask_v[:] = jnp.where(patch > 0.0, ones16, zero16)

        pltpu.async_copy(
            feats_hbm.at[n0, pl.ds(_CH, _CH), s0, :], fbufB, semB)

        def _process(cg, fbuf, fsem, outb, osem, have_pending_out):
            pltpu.make_async_copy(
                feats_hbm.at[n0, pl.ds(0, _CH), s0, :], fbuf, fsem).wait()

            for r in range(16):
                for b in sum0 + sum1:
                    b[r, :] = zero16
                for b in max0 + max1:
                    b[r, :] = neg100

            def kloop(j, ksp):
                labv = lab_v[pl.ds(j * 16, 16)]
                vmv = vm_v[pl.ds(j * 16, 16)]
                for u in range(16):
                    lsp = _splat_lane(labv, u)
                    msp = _splat_lane(vmv, u)
                    v0 = plsc.load_gather(fbuf, [iota, ksp])
                    v1 = plsc.load_gather(fbuf, [iota16, ksp])
                    mb0 = max0[u % 4]
                    mb1 = max1[u % 4]
                    c0 = plsc.load_gather(mb0, [iota, lsp])
                    c1 = plsc.load_gather(mb1, [iota, lsp])
                    plsc.store_scatter(mb0, [iota, lsp], jnp.maximum(c0, v0))
                    plsc.store_scatter(mb1, [iota, lsp], jnp.maximum(c1, v1))
                    plsc.addupdate_scatter(sum0[u % 2], [iota, lsp], v0 * msp)
                    plsc.addupdate_scatter(sum1[u % 2], [iota, lsp], v1 * msp)
                    ksp = ksp + 1
                return ksp
            lax.fori_loop(0, _K // 16, kloop, jnp.zeros((16,), jnp.int32))

            @pl.when(cg + 2 < _NCHUNK)
            def _():
                pltpu.async_copy(
                    feats_hbm.at[n0, pl.ds((cg + 2) * _CH, _CH), s0, :],
                    fbuf, fsem)

            @pl.when(have_pending_out)
            def _():
                pltpu.make_async_copy(
                    outb, out_hbm.at[n0, pl.ds(0, _CH), s0, :], osem).wait()

            recipv = recip_v[:]
            maskv = mask_v[:]
            for ch in range(16):
                srow = sum0[0][ch, :] + sum0[1][ch, :]
                mrow = jnp.maximum(
                    jnp.maximum(max0[0][ch, :], max0[1][ch, :]),
                    jnp.maximum(max0[2][ch, :], max0[3][ch, :]))
                outb[ch, :] = srow * recipv + mrow * maskv
                srow1 = sum1[0][ch, :] + sum1[1][ch, :]
                mrow1 = jnp.maximum(
                    jnp.maximum(max1[0][ch, :], max1[1][ch, :]),
                    jnp.maximum(max1[2][ch, :], max1[3][ch, :]))
                outb[ch + 16, :] = srow1 * recipv + mrow1 * maskv
            pltpu.async_copy(
                outb, out_hbm.at[n0, pl.ds(cg * _CH, _CH), s0, :], osem)

        def chunk_pair(g, carry):
            _process(2 * g, fbufA, semA, outbA, semOA, g > 0)
            _process(2 * g + 1, fbufB, semB, outbB, semOB, g > 0)
            return carry
        lax.fori_loop(0, _NCHUNK // 2, chunk_pair, 0)

        pltpu.make_async_copy(
            outbA, out_hbm.at[n0, pl.ds(0, _CH), s0, :], semOA).wait()
        pltpu.make_async_copy(
            outbB, out_hbm.at[n0, pl.ds(0, _CH), s0, :], semOB).wait()


@jax.jit
def kernel(feats, part_labels, valid_mask):
    n, c, s, k = feats.shape
    vmf = valid_mask.astype(jnp.float32)

    mesh = plsc.VectorSubcoreMesh(core_axis_name="c", subcore_axis_name="s")
    f32 = jnp.float32
    bb = pltpu.VMEM((16, _P), f32)
    run = functools.partial(
        pl.kernel,
        mesh=mesh,
        compiler_params=pltpu.CompilerParams(needs_layout_passes=False),
        out_type=jax.ShapeDtypeStruct((n, c, s, _P), f32),
        scratch_types=[
            pltpu.VMEM((_CH, _K), f32),
            pltpu.VMEM((_CH, _K), f32),
            pltpu.VMEM((_K,), jnp.int32),
            pltpu.VMEM((_K,), f32),
            bb, bb,
            bb, bb, bb, bb,
            bb, bb,
            bb, bb, bb, bb,
            bb, bb, bb, bb,
            pltpu.VMEM((_P,), f32),
            pltpu.VMEM((_P,), f32),
            pltpu.VMEM((_CH, _P), f32),
            pltpu.VMEM((_CH, _P), f32),
            pltpu.SemaphoreType.DMA,
            pltpu.SemaphoreType.DMA,
            pltpu.SemaphoreType.DMA,
            pltpu.SemaphoreType.DMA,
        ],
    )(_sc_body)
    return run(feats, part_labels, vmf)


# speedup vs baseline: 4.5979x; 4.5979x over previous
import functools

import numpy as np
import jax
import jax.numpy as jnp
from jax import lax
from jax.experimental import pallas as pl
from jax.experimental.pallas import tpu as pltpu
from jax.experimental.pallas import tpu_sc as plsc

_P = 16
_N, _C, _S, _K = 16, 256, 8, 1024
_NW = 32
_ROWS = _N * _S
_RPW = _ROWS // _NW
_CH = 16
_NCHUNK = _C // _CH


def _perm16(vec, idxv):
    idx = idxv.reshape(16, 1)
    dnums = lax.GatherDimensionNumbers(
        offset_dims=(), collapsed_slice_dims=(0,), start_index_map=(0,))
    return lax.gather(vec, idx, dnums, (1,),
                      mode=lax.GatherScatterMode.PROMISE_IN_BOUNDS)


def _splat_lane(vec, u):
    return _perm16(vec, jnp.full((16,), u, jnp.int32))


def _transpose16(vr, iota):
    for s in (1, 2, 4, 8):
        perm = jnp.bitwise_xor(iota, s)
        mask = jnp.bitwise_and(iota, s) != 0
        nv = list(vr)
        for i in range(16):
            if i & s:
                continue
            a, b = vr[i], vr[i | s]
            pa = _perm16(a, perm)
            pb = _perm16(b, perm)
            nv[i] = jnp.where(mask, pb, a)
            nv[i | s] = jnp.where(mask, b, pa)
        vr = nv
    return vr


def _sc_body(feats_hbm, lab_hbm, vm_hbm, out_hbm,
             fbufA, fbufB, lab_v, vm_v,
             sumb0, sumb1, maxb0, maxb1, maxb2, maxb3,
             cntb0, cntb1, patchb0, patchb1,
             recip_v, mask_v, outbA, outbB,
             semA, semB, semOA, semOB):
    cid = lax.axis_index("c")
    sid = lax.axis_index("s")
    wid = sid * 2 + cid
    iota = lax.iota(jnp.int32, 16)
    zero16 = jnp.zeros((16,), jnp.float32)
    neg100 = jnp.full((16,), -100.0, jnp.float32)
    ones16 = jnp.ones((16,), jnp.float32)
    sumbs = (sumb0, sumb1)
    maxbs = (maxb0, maxb1, maxb2, maxb3)

    for ri in range(_RPW):
        row = wid * _RPW + ri
        n0 = row // _S
        s0 = lax.rem(row, _S)

        pltpu.async_copy(
            feats_hbm.at[n0, pl.ds(0, _CH), s0, :], fbufA, semA)

        pltpu.sync_copy(lab_hbm.at[n0, s0], lab_v)
        pltpu.sync_copy(vm_hbm.at[n0, s0], vm_v)

        for r in range(16):
            cntb0[r, :] = zero16
            cntb1[r, :] = zero16
            patchb0[r, :] = zero16
            patchb1[r, :] = zero16

        def cbody(j, carry):
            for u in range(4):
                jj = j * 4 + u
                labv = lab_v[pl.ds(jj * 16, 16)]
                vmv = vm_v[pl.ds(jj * 16, 16)]
                cb = (cntb0, cntb1)[u % 2]
                pb = (patchb0, patchb1)[u % 2]
                plsc.addupdate_scatter(cb, [iota, labv], vmv)
                plsc.addupdate_scatter(pb, [iota, labv], ones16)
            return carry
        lax.fori_loop(0, _K // 64, cbody, 0)

        cnt = cntb0[0, :] + cntb1[0, :]
        patch = patchb0[0, :] + patchb1[0, :]
        for r in range(1, 16):
            cnt = cnt + cntb0[r, :] + cntb1[r, :]
            patch = patch + patchb0[r, :] + patchb1[r, :]
        recip_v[:] = 1.0 / jnp.maximum(cnt, 1.0)
        mask_v[:] = jnp.where(patch > 0.0, ones16, zero16)

        pltpu.async_copy(
            feats_hbm.at[n0, pl.ds(_CH, _CH), s0, :], fbufB, semB)

        def _process(cg, fbuf, fsem, outb, osem, have_pending_out):
            pltpu.make_async_copy(
                feats_hbm.at[n0, pl.ds(0, _CH), s0, :], fbuf, fsem).wait()

            for r in range(16):
                sumb0[r, :] = zero16
                sumb1[r, :] = zero16
                maxb0[r, :] = neg100
                maxb1[r, :] = neg100
                maxb2[r, :] = neg100
                maxb3[r, :] = neg100

            def kbody(j, carry):
                vr = [fbuf[ch, pl.ds(j * 16, 16)] for ch in range(16)]
                vt = _transpose16(vr, iota)
                labv = lab_v[pl.ds(j * 16, 16)]
                vmv = vm_v[pl.ds(j * 16, 16)]
                for u in range(16):
                    lsp = _splat_lane(labv, u)
                    msp = _splat_lane(vmv, u)
                    v = vt[u]
                    mb = maxbs[u % 4]
                    cur = plsc.load_gather(mb, [lsp, iota])
                    plsc.store_scatter(mb, [lsp, iota], jnp.maximum(cur, v))
                    plsc.addupdate_scatter(sumbs[u % 2], [lsp, iota],
                                           v * msp)
                return carry
            lax.fori_loop(0, _K // 16, kbody, 0)

            @pl.when(cg + 2 < _NCHUNK)
            def _():
                pltpu.async_copy(
                    feats_hbm.at[n0, pl.ds((cg + 2) * _CH, _CH), s0, :],
                    fbuf, fsem)

            @pl.when(have_pending_out)
            def _():
                pltpu.make_async_copy(
                    outb, out_hbm.at[n0, pl.ds(0, _CH), s0, :], osem).wait()

            recipv = recip_v[:]
            maskv = mask_v[:]
            for p in range(_P):
                srow = sumb0[p, :] + sumb1[p, :]
                mrow = jnp.maximum(jnp.maximum(maxb0[p, :], maxb1[p, :]),
                                   jnp.maximum(maxb2[p, :], maxb3[p, :]))
                rp = _splat_lane(recipv, p)
                mp = _splat_lane(maskv, p)
                col = srow * rp + mrow * mp
                pidx = jnp.full((16,), p, jnp.int32)
                plsc.store_scatter(outb, [iota, pidx], col)
            pltpu.async_copy(
                outb, out_hbm.at[n0, pl.ds(cg * _CH, _CH), s0, :], osem)

        def chunk_pair(g, carry):
            _process(2 * g, fbufA, semA, outbA, semOA, g > 0)
            _process(2 * g + 1, fbufB, semB, outbB, semOB, g > 0)
            return carry
        lax.fori_loop(0, _NCHUNK // 2, chunk_pair, 0)

        pltpu.make_async_copy(
            outbA, out_hbm.at[n0, pl.ds(0, _CH), s0, :], semOA).wait()
        pltpu.make_async_copy(
            outbB, out_hbm.at[n0, pl.ds(0, _CH), s0, :], semOB).wait()


@jax.jit
def kernel(feats, part_labels, valid_mask):
    n, c, s, k = feats.shape
    vmf = valid_mask.astype(jnp.float32)

    mesh = plsc.VectorSubcoreMesh(core_axis_name="c", subcore_axis_name="s")
    f32 = jnp.float32
    bb = pltpu.VMEM((_P, 16), f32)
    run = functools.partial(
        pl.kernel,
        mesh=mesh,
        compiler_params=pltpu.CompilerParams(needs_layout_passes=False),
        out_type=jax.ShapeDtypeStruct((n, c, s, _P), f32),
        scratch_types=[
            pltpu.VMEM((_CH, _K), f32),
            pltpu.VMEM((_CH, _K), f32),
            pltpu.VMEM((_K,), jnp.int32),
            pltpu.VMEM((_K,), f32),
            bb, bb,
            bb, bb, bb, bb,
            bb, bb, bb, bb,
            pltpu.VMEM((_P,), f32),
            pltpu.VMEM((_P,), f32),
            pltpu.VMEM((_CH, _P), f32),
            pltpu.VMEM((_CH, _P), f32),
            pltpu.SemaphoreType.DMA,
            pltpu.SemaphoreType.DMA,
            pltpu.SemaphoreType.DMA,
            pltpu.SemaphoreType.DMA,
        ],
    )(_sc_body)
    return run(feats, part_labels, vmf)


# speedup vs baseline: 6.6164x; 1.4390x over previous
import functools

import jax
import jax.numpy as jnp
from jax import lax
from jax.experimental import pallas as pl
from jax.experimental.pallas import tpu as pltpu
from jax.experimental.pallas import tpu_sc as plsc

_P = 16
_N, _C, _S, _K = 16, 256, 8, 1024
_NW = 32
_ROWS = _N * _S
_RPW = _ROWS // _NW
_CH = 16
_NCHUNK = _C // _CH


def _perm16(vec, idxv):
    idx = idxv.reshape(16, 1)
    dnums = lax.GatherDimensionNumbers(
        offset_dims=(), collapsed_slice_dims=(0,), start_index_map=(0,))
    return lax.gather(vec, idx, dnums, (1,),
                      mode=lax.GatherScatterMode.PROMISE_IN_BOUNDS)


def _splat_lane(vec, u):
    return _perm16(vec, jnp.full((16,), u, jnp.int32))


def _transpose16(vr, iota):
    for s in (1, 2, 4, 8):
        perm = jnp.bitwise_xor(iota, s)
        mask = jnp.bitwise_and(iota, s) != 0
        nv = list(vr)
        for i in range(16):
            if i & s:
                continue
            a, b = vr[i], vr[i | s]
            pa = _perm16(a, perm)
            pb = _perm16(b, perm)
            nv[i] = jnp.where(mask, pb, a)
            nv[i | s] = jnp.where(mask, b, pa)
        vr = nv
    return vr


def _sc_body(feats_hbm, lab_hbm, vm_hbm, out_hbm,
             fbufA, fbufB, lab_v, vm_v,
             sumb0, sumb1, maxb0, maxb1, maxb2, maxb3,
             cntb0, cntb1, patchb0, patchb1,
             recip_v, mask_v, outbA, outbB,
             semA, semB, semOA, semOB):
    cid = lax.axis_index("c")
    sid = lax.axis_index("s")
    wid = sid * 2 + cid
    iota = lax.iota(jnp.int32, 16)
    zero16 = jnp.zeros((16,), jnp.float32)
    neg100 = jnp.full((16,), -100.0, jnp.float32)
    ones16 = jnp.ones((16,), jnp.float32)
    sumbs = (sumb0, sumb1)
    maxbs = (maxb0, maxb1, maxb2, maxb3)

    for ri in range(_RPW):
        row = wid * _RPW + ri
        n0 = row // _S
        s0 = lax.rem(row, _S)

        pltpu.async_copy(
            feats_hbm.at[n0, pl.ds(0, _CH), s0, :], fbufA, semA)

        pltpu.sync_copy(lab_hbm.at[n0, s0], lab_v)
        pltpu.sync_copy(vm_hbm.at[n0, s0], vm_v)

        for r in range(16):
            cntb0[r, :] = zero16
            cntb1[r, :] = zero16
            patchb0[r, :] = zero16
            patchb1[r, :] = zero16

        def cbody(j, carry):
            for u in range(4):
                jj = j * 4 + u
                labv = lab_v[pl.ds(jj * 16, 16)]
                vmv = vm_v[pl.ds(jj * 16, 16)]
                cb = (cntb0, cntb1)[u % 2]
                pb = (patchb0, patchb1)[u % 2]
                plsc.addupdate_scatter(cb, [iota, labv], vmv)
                plsc.addupdate_scatter(pb, [iota, labv], ones16)
            return carry
        lax.fori_loop(0, _K // 64, cbody, 0)

        cnt = cntb0[0, :] + cntb1[0, :]
        patch = patchb0[0, :] + patchb1[0, :]
        for r in range(1, 16):
            cnt = cnt + cntb0[r, :] + cntb1[r, :]
            patch = patch + patchb0[r, :] + patchb1[r, :]
        recip_v[:] = 1.0 / jnp.maximum(cnt, 1.0)
        mask_v[:] = jnp.where(patch > 0.0, ones16, zero16)

        pltpu.async_copy(
            feats_hbm.at[n0, pl.ds(_CH, _CH), s0, :], fbufB, semB)

        def _process(cg, fbuf, fsem, outb, osem, have_pending_out):
            pltpu.make_async_copy(
                feats_hbm.at[n0, pl.ds(0, _CH), s0, :], fbuf, fsem).wait()

            for r in range(16):
                sumb0[r, :] = zero16
                sumb1[r, :] = zero16
                maxb0[r, :] = neg100
                maxb1[r, :] = neg100
                maxb2[r, :] = neg100
                maxb3[r, :] = neg100

            def kbody(j, carry):
                vr = [fbuf[ch, pl.ds(j * 16, 16)] for ch in range(16)]
                vt = _transpose16(vr, iota)
                labv = lab_v[pl.ds(j * 16, 16)]
                vmv = vm_v[pl.ds(j * 16, 16)]
                for g in range(4):
                    us = range(g * 4, g * 4 + 4)
                    lsps = [_splat_lane(labv, u) for u in us]
                    msps = [_splat_lane(vmv, u) for u in us]
                    curs = [plsc.load_gather(maxbs[i], [lsps[i], iota])
                            for i in range(4)]
                    for i, u in enumerate(us):
                        v = vt[u]
                        plsc.store_scatter(maxbs[i], [lsps[i], iota],
                                           jnp.maximum(curs[i], v))
                        plsc.addupdate_scatter(sumbs[u % 2],
                                               [lsps[i], iota], v * msps[i])
                return carry
            lax.fori_loop(0, _K // 16, kbody, 0)

            @pl.when(cg + 2 < _NCHUNK)
            def _():
                pltpu.async_copy(
                    feats_hbm.at[n0, pl.ds((cg + 2) * _CH, _CH), s0, :],
                    fbuf, fsem)

            @pl.when(have_pending_out)
            def _():
                pltpu.make_async_copy(
                    outb, out_hbm.at[n0, pl.ds(0, _CH), s0, :], osem).wait()

            recipv = recip_v[:]
            maskv = mask_v[:]
            for p in range(_P):
                srow = sumb0[p, :] + sumb1[p, :]
                mrow = jnp.maximum(jnp.maximum(maxb0[p, :], maxb1[p, :]),
                                   jnp.maximum(maxb2[p, :], maxb3[p, :]))
                rp = _splat_lane(recipv, p)
                mp = _splat_lane(maskv, p)
                col = srow * rp + mrow * mp
                pidx = jnp.full((16,), p, jnp.int32)
                plsc.store_scatter(outb, [iota, pidx], col)
            pltpu.async_copy(
                outb, out_hbm.at[n0, pl.ds(cg * _CH, _CH), s0, :], osem)

        def chunk_pair(g, carry):
            _process(2 * g, fbufA, semA, outbA, semOA, g > 0)
            _process(2 * g + 1, fbufB, semB, outbB, semOB, g > 0)
            return carry
        lax.fori_loop(0, _NCHUNK // 2, chunk_pair, 0)

        pltpu.make_async_copy(
            outbA, out_hbm.at[n0, pl.ds(0, _CH), s0, :], semOA).wait()
        pltpu.make_async_copy(
            outbB, out_hbm.at[n0, pl.ds(0, _CH), s0, :], semOB).wait()


@jax.jit
def kernel(feats, part_labels, valid_mask):
    n, c, s, k = feats.shape
    vmf = valid_mask.astype(jnp.float32)

    mesh = plsc.VectorSubcoreMesh(core_axis_name="c", subcore_axis_name="s")
    f32 = jnp.float32
    bb = pltpu.VMEM((_P, 16), f32)
    run = functools.partial(
        pl.kernel,
        mesh=mesh,
        compiler_params=pltpu.CompilerParams(needs_layout_passes=False),
        out_type=jax.ShapeDtypeStruct((n, c, s, _P), f32),
        scratch_types=[
            pltpu.VMEM((_CH, _K), f32),
            pltpu.VMEM((_CH, _K), f32),
            pltpu.VMEM((_K,), jnp.int32),
            pltpu.VMEM((_K,), f32),
            bb, bb,
            bb, bb, bb, bb,
            bb, bb, bb, bb,
            pltpu.VMEM((_P,), f32),
            pltpu.VMEM((_P,), f32),
            pltpu.VMEM((_CH, _P), f32),
            pltpu.VMEM((_CH, _P), f32),
            pltpu.SemaphoreType.DMA,
            pltpu.SemaphoreType.DMA,
            pltpu.SemaphoreType.DMA,
            pltpu.SemaphoreType.DMA,
        ],
    )(_sc_body)
    return run(feats, part_labels, vmf)
